# baseline (speedup 1.0000x reference)
import functools

import jax
import jax.numpy as jnp
from jax import lax
from jax.experimental import pallas as pl
from jax.experimental.pallas import tpu as pltpu
from jax.experimental.pallas import tpu_sc as plsc

NC = 2
NS = 16
NW = NC * NS
L = 16
N = 64 * 1024
C = 128
T = N // NW // C
D_OBS = 256
D_ACT = 16
D_OUT = D_OBS + D_ACT + 1

_mesh = plsc.VectorSubcoreMesh(core_axis_name="c", subcore_axis_name="s")


@functools.partial(
    pl.kernel,
    mesh=_mesh,
    compiler_params=pltpu.CompilerParams(use_tc_tiling_on_sc=False),
    out_type=jax.ShapeDtypeStruct((N, D_OUT), jnp.float32),
    scratch_types=[
        pltpu.VMEM((T, C), jnp.int32),
        pltpu.VMEM((C, D_OBS), jnp.float32),
        pltpu.VMEM((C, D_ACT), jnp.float32),
        pltpu.SemaphoreType.DMA,
        pltpu.SemaphoreType.DMA,
    ],
)
def _gather_obs_act(obs_hbm, act_hbm, idx_hbm, out_hbm,
                    idx_v, obs_v, act_v, s_obs, s_act):
    wid = lax.axis_index("s") * NC + lax.axis_index("c")
    pltpu.sync_copy(idx_hbm.at[pl.ds(wid * T, T)], idx_v)
    for t in range(T):
        row0 = wid * T * C + t * C
        cp_obs = pltpu.async_copy(obs_hbm.at[idx_v.at[t]], obs_v, s_obs)
        cp_act = pltpu.async_copy(act_hbm.at[idx_v.at[t]], act_v, s_act)
        cp_obs.wait()
        pltpu.sync_copy(obs_v, out_hbm.at[pl.ds(row0, C), pl.ds(0, D_OBS)])
        cp_act.wait()
        pltpu.sync_copy(act_v, out_hbm.at[pl.ds(row0, C), pl.ds(D_OBS, D_ACT)])


@functools.partial(
    pl.kernel,
    mesh=_mesh,
    compiler_params=pltpu.CompilerParams(use_tc_tiling_on_sc=False),
    out_type=(),
    scratch_types=[
        pltpu.VMEM((T, C), jnp.int32),
        pltpu.VMEM((T, C), jnp.int32),
        pltpu.VMEM((C,), jnp.float32),
        pltpu.SemaphoreType.DMA,
    ],
)
def _scatter_rew(rew_hbm, idx_hbm, out_hbm,
                 idx_v, pos_v, rew_g, s_rew):
    wid = lax.axis_index("s") * NC + lax.axis_index("c")
    pltpu.sync_copy(idx_hbm.at[pl.ds(wid * T, T)], idx_v)
    for t in range(T):
        for g in range(C // L):
            row = wid * T * C + t * C + g * L
            pos_v[t, pl.ds(g * L, L)] = (
                (lax.iota(jnp.int32, L) + row) * D_OUT + (D_OUT - 1))
    for t in range(T):
        pltpu.async_copy(rew_hbm.at[idx_v.at[t]], rew_g, s_rew).wait()
        pltpu.async_copy(rew_g, out_hbm.at[pos_v.at[t]], s_rew).wait()


def kernel(obs, actions, rewards, batch_indices):
    obs_f = obs.reshape(N, D_OBS)
    act_f = actions.reshape(N, D_ACT)
    rew_f = rewards.reshape(N)
    idx = batch_indices.reshape(N // C, C)
    out = _gather_obs_act(obs_f, act_f, idx)
    out_ref = jax.new_ref(out.reshape(N * D_OUT))
    _scatter_rew(rew_f, idx, out_ref)
    return out_ref[...].reshape(64, 1024, D_OUT)


# speedup vs baseline: 1.0177x; 1.0177x over previous
import functools

import jax
import jax.numpy as jnp
from jax import lax
from jax.experimental import pallas as pl
from jax.experimental.pallas import tpu as pltpu
from jax.experimental.pallas import tpu_sc as plsc

NC = 2
NS = 16
NW = NC * NS
L = 16
N = 64 * 1024
C = 128
T = N // NW // C
D_OBS = 256
D_ACT = 16
D_OUT = D_OBS + D_ACT + 1

_mesh = plsc.VectorSubcoreMesh(core_axis_name="c", subcore_axis_name="s")


@functools.partial(
    pl.kernel,
    mesh=_mesh,
    compiler_params=pltpu.CompilerParams(use_tc_tiling_on_sc=False),
    out_type=jax.ShapeDtypeStruct((N, D_OUT), jnp.float32),
    scratch_types=[
        pltpu.VMEM((T, C), jnp.int32),
        pltpu.VMEM((C, D_OBS), jnp.float32),
        pltpu.VMEM((C, D_OBS), jnp.float32),
        pltpu.VMEM((C, D_OBS), jnp.float32),
        pltpu.VMEM((C, D_ACT), jnp.float32),
        pltpu.VMEM((C, D_ACT), jnp.float32),
        pltpu.VMEM((C, D_ACT), jnp.float32),
        pltpu.SemaphoreType.DMA,
        pltpu.SemaphoreType.DMA,
        pltpu.SemaphoreType.DMA,
        pltpu.SemaphoreType.DMA,
        pltpu.SemaphoreType.DMA,
        pltpu.SemaphoreType.DMA,
    ],
)
def _gather_obs_act(obs_hbm, act_hbm, idx_hbm, out_hbm,
                    idx_v, o0, o1, o2, a0, a1, a2, sg0, sg1, sg2, sw0, sw1, sw2):
    NB = 3
    obufs = (o0, o1, o2)
    abufs = (a0, a1, a2)
    gsem = (sg0, sg1, sg2)
    wsem = (sw0, sw1, sw2)
    wid = lax.axis_index("s") * NC + lax.axis_index("c")
    pltpu.sync_copy(idx_hbm.at[pl.ds(wid * T, T)], idx_v)

    gcp = {}
    wcp = [None] * NB

    def fire(t):
        b = t % NB
        cpo = pltpu.async_copy(obs_hbm.at[idx_v.at[t]], obufs[b], gsem[b])
        cpa = pltpu.async_copy(act_hbm.at[idx_v.at[t]], abufs[b], gsem[b])
        gcp[t] = (cpo, cpa)

    fire(0)
    for t in range(T):
        b = t % NB
        tn = t + 1
        if tn < T:
            bn = tn % NB
            if wcp[bn] is not None:
                for w in wcp[bn]:
                    w.wait()
                wcp[bn] = None
            fire(tn)
        cpo, cpa = gcp.pop(t)
        cpo.wait()
        cpa.wait()
        row0 = wid * T * C + t * C
        wo = pltpu.async_copy(
            obufs[b], out_hbm.at[pl.ds(row0, C), pl.ds(0, D_OBS)], wsem[b])
        wa = pltpu.async_copy(
            abufs[b], out_hbm.at[pl.ds(row0, C), pl.ds(D_OBS, D_ACT)], wsem[b])
        wcp[b] = (wo, wa)
    for ws in wcp:
        if ws is not None:
            for w in ws:
                w.wait()


@functools.partial(
    pl.kernel,
    mesh=_mesh,
    compiler_params=pltpu.CompilerParams(use_tc_tiling_on_sc=False),
    out_type=(),
    scratch_types=[
        pltpu.VMEM((T, C), jnp.int32),
        pltpu.VMEM((T, C), jnp.int32),
        pltpu.VMEM((C,), jnp.float32),
        pltpu.SemaphoreType.DMA,
    ],
)
def _scatter_rew(rew_hbm, idx_hbm, out_hbm,
                 idx_v, pos_v, rew_g, s_rew):
    wid = lax.axis_index("s") * NC + lax.axis_index("c")
    pltpu.sync_copy(idx_hbm.at[pl.ds(wid * T, T)], idx_v)
    for t in range(T):
        for g in range(C // L):
            row = wid * T * C + t * C + g * L
            pos_v[t, pl.ds(g * L, L)] = (
                (lax.iota(jnp.int32, L) + row) * D_OUT + (D_OUT - 1))
    for t in range(T):
        pltpu.async_copy(rew_hbm.at[idx_v.at[t]], rew_g, s_rew).wait()
        pltpu.async_copy(rew_g, out_hbm.at[pos_v.at[t]], s_rew).wait()


def kernel(obs, actions, rewards, batch_indices):
    obs_f = obs.reshape(N, D_OBS)
    act_f = actions.reshape(N, D_ACT)
    rew_f = rewards.reshape(N)
    idx = batch_indices.reshape(N // C, C)
    out = _gather_obs_act(obs_f, act_f, idx)
    out_ref = jax.new_ref(out.reshape(N * D_OUT))
    _scatter_rew(rew_f, idx, out_ref)
    return out_ref[...].reshape(64, 1024, D_OUT)


# speedup vs baseline: 1.6928x; 1.6633x over previous
import functools

import jax
import jax.numpy as jnp
from jax import lax
from jax.experimental import pallas as pl
from jax.experimental.pallas import tpu as pltpu
from jax.experimental.pallas import tpu_sc as plsc

NC = 2
NS = 16
NW = NC * NS
N = 64 * 1024
C = 128
T = N // NW // C
D_OBS = 256
D_ACT = 16
D_AR = D_ACT + 1
D_OUT = D_OBS + D_AR

_mesh = plsc.VectorSubcoreMesh(core_axis_name="c", subcore_axis_name="s")


@functools.partial(
    pl.kernel,
    mesh=_mesh,
    compiler_params=pltpu.CompilerParams(use_tc_tiling_on_sc=False),
    out_type=jax.ShapeDtypeStruct((N, D_OUT), jnp.float32),
    scratch_types=[
        pltpu.VMEM((T, C), jnp.int32),
        pltpu.VMEM((C, D_OBS), jnp.float32),
        pltpu.VMEM((C, D_OBS), jnp.float32),
        pltpu.VMEM((C, D_OBS), jnp.float32),
        pltpu.VMEM((C, D_AR), jnp.float32),
        pltpu.VMEM((C, D_AR), jnp.float32),
        pltpu.VMEM((C, D_AR), jnp.float32),
        pltpu.SemaphoreType.DMA,
        pltpu.SemaphoreType.DMA,
        pltpu.SemaphoreType.DMA,
        pltpu.SemaphoreType.DMA,
        pltpu.SemaphoreType.DMA,
        pltpu.SemaphoreType.DMA,
    ],
)
def _gather_all(obs_hbm, ar_hbm, idx_hbm, out_hbm,
                idx_v, o0, o1, o2, a0, a1, a2,
                sg0, sg1, sg2, sw0, sw1, sw2):
    NB = 3
    obufs = (o0, o1, o2)
    abufs = (a0, a1, a2)
    gsem = (sg0, sg1, sg2)
    wsem = (sw0, sw1, sw2)
    wid = lax.axis_index("s") * NC + lax.axis_index("c")
    pltpu.sync_copy(idx_hbm.at[pl.ds(wid * T, T)], idx_v)

    gcp = {}
    wcp = [None] * NB

    def fire(t):
        b = t % NB
        cpo = pltpu.async_copy(obs_hbm.at[idx_v.at[t]], obufs[b], gsem[b])
        cpa = pltpu.async_copy(ar_hbm.at[idx_v.at[t]], abufs[b], gsem[b])
        gcp[t] = (cpo, cpa)

    fire(0)
    for t in range(T):
        b = t % NB
        tn = t + 1
        if tn < T:
            bn = tn % NB
            if wcp[bn] is not None:
                for w in wcp[bn]:
                    w.wait()
                wcp[bn] = None
            fire(tn)
        cpo, cpa = gcp.pop(t)
        cpo.wait()
        cpa.wait()
        row0 = wid * T * C + t * C
        wo = pltpu.async_copy(
            obufs[b], out_hbm.at[pl.ds(row0, C), pl.ds(0, D_OBS)], wsem[b])
        wa = pltpu.async_copy(
            abufs[b], out_hbm.at[pl.ds(row0, C), pl.ds(D_OBS, D_AR)], wsem[b])
        wcp[b] = (wo, wa)
    for ws in wcp:
        if ws is not None:
            for w in ws:
                w.wait()


def kernel(obs, actions, rewards, batch_indices):
    obs_f = obs.reshape(N, D_OBS)
    ar_f = jnp.concatenate(
        [actions.reshape(N, D_ACT), rewards.reshape(N, 1)], axis=-1)
    idx = batch_indices.reshape(N // C, C)
    out = _gather_all(obs_f, ar_f, idx)
    return out.reshape(64, 1024, D_OUT)
